# baseline (speedup 1.0000x reference)
import jax
import jax.numpy as jnp
from jax import lax
from jax.experimental import pallas as pl
from jax.experimental.pallas import tpu as pltpu
from jax.experimental.pallas import tpu_sc as plsc

N_FIELDS = 26
NUM_CATEGORIES = 1000
LATENT_DIM = 8
DIM = 64
BATCH = 16384
EPS = 1e-4

NON_CATEG = DIM - N_FIELDS
OUT_COLS = NON_CATEG + N_FIELDS * LATENT_DIM
GROUP = 16

QUARTERS = 8
TASK_ROWS = BATCH // QUARTERS
LANES = 16
TASK_GROUPS = TASK_ROWS // LANES

PREP_BLOCK = 2048
CONCAT_BLOCK = 2048


KPAD = 1024


def _prep_body(xt_ref, raw_ref, tab_ref, idxt_ref):
  @pl.when(pl.program_id(0) == 0)
  def _():
    t = EPS + (1.0 - 2.0 * EPS) * jax.nn.sigmoid(raw_ref[...])
    tp = jnp.concatenate(
        [t, jnp.zeros((N_FIELDS, LATENT_DIM, KPAD - NUM_CATEGORIES),
                      jnp.float32)], axis=-1)
    tab_ref[...] = tp.reshape(N_FIELDS, LATENT_DIM, KPAD // 128, 128)

  idxt_ref[...] = xt_ref[:N_FIELDS, :].astype(jnp.int32).reshape(
      N_FIELDS, PREP_BLOCK // 128, 128)


def _sc_body(tab2, idxt_hbm, emba_hbm, embb_hbm, tab_v0, tab_v1, idx_v0,
             idx_v1, obuf0, obuf1, sg0, sg1, sw0, sw1):
  wid = lax.axis_index("s") * 2 + lax.axis_index("c")
  tab_vs = (tab_v0, tab_v1)
  idx_vs = (idx_v0, idx_v1)
  obufs = (obuf0, obuf1)
  sg = (sg0, sg1)
  sw = (sw0, sw1)
  lsplats = [jnp.full((LANES,), l, jnp.int32) for l in range(LATENT_DIM)]
  pending = [None, None]

  defs = []
  for i in range(4):
    a = 4 * wid + i
    defs.append((a % GROUP, a // GROUP, emba_hbm))
  for i in range(2):
    b = 2 * wid + i
    defs.append((GROUP + b % 10, b // 10, embb_hbm))
  b6 = 64 + wid
  defs.append((GROUP + b6 % 10, jnp.minimum(b6 // 10, QUARTERS - 1),
               embb_hbm))

  def fire_stage(k):
    p = k % 2
    f, q, _ = defs[k]
    c1 = pltpu.async_copy(tab2.at[f], tab_vs[p], sg[p])
    c2 = pltpu.async_copy(
        idxt_hbm.at[f, pl.ds(pl.multiple_of(q * (TASK_ROWS // 128), 8),
                             TASK_ROWS // 128)],
        idx_vs[p], sg[p])
    return (c1, c2)

  def compute(k):
    p = k % 2
    tab_v, idx_v, obuf = tab_vs[p], idx_vs[p], obufs[p]

    @plsc.parallel_loop(0, TASK_GROUPS, unroll=4)
    def _(g):
      gr = g // 8
      gc = (g % 8) * LANES
      iv = idx_v[gr, pl.ds(gc, LANES)]
      hi = lax.shift_right_logical(iv, 7)
      lo = lax.bitwise_and(iv, 127)
      vs = [plsc.load_gather(tab_v, [lsplats[l], hi, lo])
            for l in range(LATENT_DIM)]
      for l in range(LATENT_DIM):
        obuf[l, gr, pl.ds(gc, LANES)] = vs[l]

  def fire_write(k):
    p = k % 2
    f, q, emb_hbm = defs[k]
    row0 = pl.multiple_of((f % GROUP) * LATENT_DIM, 8)
    col0 = pl.multiple_of(q * (TASK_ROWS // 128), 8)
    return pltpu.async_copy(
        obufs[p],
        emb_hbm.at[pl.ds(row0, LATENT_DIM), pl.ds(col0, TASK_ROWS // 128)],
        sw[p],
    )

  stage_pending = {0: fire_stage(0)}
  for k in range(7):
    if k + 1 < 7:
      stage_pending[k + 1] = fire_stage(k + 1)
    for c in stage_pending.pop(k):
      c.wait()
    if k == 6:
      break
    p = k % 2
    if pending[p] is not None:
      pending[p].wait()
    compute(k)
    pending[p] = fire_write(k)

  pending[0].wait()
  pending[0] = None

  @pl.when(wid < 16)
  def _():
    compute(6)
    fire_write(6).wait()

  pending[1].wait()


def _concat_body(xt_ref, emba_ref, embb_ref, out_ref):
  out_ref[:NON_CATEG, :] = xt_ref[N_FIELDS:, :]
  ea = emba_ref[...].reshape(GROUP * LATENT_DIM, CONCAT_BLOCK)
  eb = embb_ref[...].reshape(GROUP * LATENT_DIM, CONCAT_BLOCK)
  out_ref[NON_CATEG:NON_CATEG + GROUP * LATENT_DIM, :] = ea
  out_ref[NON_CATEG + GROUP * LATENT_DIM:, :] = (
      eb[:(N_FIELDS - GROUP) * LATENT_DIM, :])


@jax.jit
def kernel(X, raw_emb_tables):
  XT = X.T
  raw_t = raw_emb_tables.transpose(0, 2, 1)
  nprep = BATCH // PREP_BLOCK
  tab2, idxt = pl.pallas_call(
      _prep_body,
      grid=(nprep,),
      in_specs=[
          pl.BlockSpec((DIM, PREP_BLOCK), lambda k: (0, k)),
          pl.BlockSpec((N_FIELDS, LATENT_DIM, NUM_CATEGORIES),
                       lambda k: (0, 0, 0)),
      ],
      out_specs=[
          pl.BlockSpec((N_FIELDS, LATENT_DIM, KPAD // 128, 128),
                       lambda k: (0, 0, 0, 0)),
          pl.BlockSpec((N_FIELDS, PREP_BLOCK // 128, 128),
                       lambda k: (0, k, 0)),
      ],
      out_shape=[
          jax.ShapeDtypeStruct((N_FIELDS, LATENT_DIM, KPAD // 128, 128),
                               jnp.float32),
          jax.ShapeDtypeStruct((N_FIELDS, BATCH // 128, 128), jnp.int32),
      ],
  )(XT, raw_t)

  mesh = plsc.VectorSubcoreMesh(core_axis_name="c", subcore_axis_name="s")
  embat, embbt = pl.kernel(
      _sc_body,
      out_type=[
          jax.ShapeDtypeStruct((GROUP * LATENT_DIM, BATCH // 128, 128),
                               jnp.float32),
          jax.ShapeDtypeStruct((GROUP * LATENT_DIM, BATCH // 128, 128),
                               jnp.float32),
      ],
      mesh=mesh,
      compiler_params=pltpu.CompilerParams(
          use_tc_tiling_on_sc=False, needs_layout_passes=False),
      scratch_types=[
          pltpu.VMEM((LATENT_DIM, KPAD // 128, 128), jnp.float32),
          pltpu.VMEM((LATENT_DIM, KPAD // 128, 128), jnp.float32),
          pltpu.VMEM((TASK_ROWS // 128, 128), jnp.int32),
          pltpu.VMEM((TASK_ROWS // 128, 128), jnp.int32),
          pltpu.VMEM((LATENT_DIM, TASK_ROWS // 128, 128), jnp.float32),
          pltpu.VMEM((LATENT_DIM, TASK_ROWS // 128, 128), jnp.float32),
          pltpu.SemaphoreType.DMA,
          pltpu.SemaphoreType.DMA,
          pltpu.SemaphoreType.DMA,
          pltpu.SemaphoreType.DMA,
      ],
  )(tab2, idxt)

  nblk = BATCH // CONCAT_BLOCK
  outt = pl.pallas_call(
      _concat_body,
      grid=(nblk,),
      in_specs=[
          pl.BlockSpec((DIM, CONCAT_BLOCK), lambda k: (0, k)),
          pl.BlockSpec((GROUP * LATENT_DIM, CONCAT_BLOCK // 128, 128),
                       lambda k: (0, k, 0)),
          pl.BlockSpec((GROUP * LATENT_DIM, CONCAT_BLOCK // 128, 128),
                       lambda k: (0, k, 0)),
      ],
      out_specs=pl.BlockSpec((OUT_COLS, CONCAT_BLOCK), lambda k: (0, k)),
      out_shape=jax.ShapeDtypeStruct((OUT_COLS, BATCH), jnp.float32),
  )(XT, embat, embbt)
  return outt.T
